# baseline (speedup 1.0000x reference)
import functools

import jax
import jax.numpy as jnp
from jax import lax
from jax.experimental import pallas as pl
from jax.experimental.pallas import tpu as pltpu
from jax.experimental.pallas import tpu_sc as plsc

NC = 2
NS = 16
NW = NC * NS

ROWS_PER_CHUNK = 32
CHUNK = ROWS_PER_CHUNK * 1024


def kernel(x, pe, position):
    B, S, D = x.shape
    M = pe.shape[0]
    n_elem = B * S * D
    rows_per_w = S // NW
    n_chunks = rows_per_w // ROWS_PER_CHUNK
    x1 = x.reshape(n_elem)
    pe1 = pe.reshape(M * D)

    mesh = plsc.VectorSubcoreMesh(core_axis_name="c", subcore_axis_name="s")

    @functools.partial(
        pl.kernel,
        mesh=mesh,
        out_type=jax.ShapeDtypeStruct((n_elem,), jnp.float32),
        scratch_types=[
            pltpu.VMEM((CHUNK,), jnp.float32),
            pltpu.VMEM((CHUNK,), jnp.float32),
            pltpu.VMEM((CHUNK,), jnp.float32),
        ],
    )
    def sc_add(x_hbm, pe_hbm, out_hbm, pbuf, xbuf, obuf):
        wid = lax.axis_index("s") * NC + lax.axis_index("c")
        for pc in range(n_chunks):
            row0 = wid * rows_per_w + pc * ROWS_PER_CHUNK
            peoff = pl.multiple_of(row0 * D, CHUNK)
            pltpu.sync_copy(pe_hbm.at[pl.ds(peoff, CHUNK)], pbuf)
            for b in range(B):
                xoff = pl.multiple_of(b * S * D + row0 * D, CHUNK)
                pltpu.sync_copy(x_hbm.at[pl.ds(xoff, CHUNK)], xbuf)

                def body(i, carry):
                    o = pl.multiple_of(i * 16, 16)
                    obuf[pl.ds(o, 16)] = xbuf[pl.ds(o, 16)] + pbuf[pl.ds(o, 16)]
                    return carry

                lax.fori_loop(0, CHUNK // 16, body, 0)
                pltpu.sync_copy(obuf, out_hbm.at[pl.ds(xoff, CHUNK)])

    out = sc_add(x1, pe1)
    return out.reshape(B, S, D)


# speedup vs baseline: 1.3808x; 1.3808x over previous
import functools

import jax
import jax.numpy as jnp
from jax import lax
from jax.experimental import pallas as pl
from jax.experimental.pallas import tpu as pltpu
from jax.experimental.pallas import tpu_sc as plsc

NC = 2
NS = 16
NW = NC * NS

ROWS_PER_CHUNK = 32
CHUNK = ROWS_PER_CHUNK * 1024


def kernel(x, pe, position):
    B, S, D = x.shape
    M = pe.shape[0]
    n_elem = B * S * D
    rows_per_w = S // NW
    n_chunks = rows_per_w // ROWS_PER_CHUNK
    x1 = x.reshape(n_elem)
    pe1 = pe.reshape(M * D)

    mesh = plsc.VectorSubcoreMesh(core_axis_name="c", subcore_axis_name="s")

    @functools.partial(
        pl.kernel,
        mesh=mesh,
        out_type=jax.ShapeDtypeStruct((n_elem,), jnp.float32),
        scratch_types=[
            pltpu.VMEM((CHUNK,), jnp.float32),
            pltpu.VMEM((CHUNK,), jnp.float32),
            pltpu.VMEM((CHUNK,), jnp.float32),
        ],
    )
    def sc_add(x_hbm, pe_hbm, out_hbm, pbuf, xbuf, obuf):
        wid = lax.axis_index("s") * NC + lax.axis_index("c")
        for pc in range(n_chunks):
            row0 = wid * rows_per_w + pc * ROWS_PER_CHUNK
            peoff = pl.multiple_of(row0 * D, CHUNK)
            pltpu.sync_copy(pe_hbm.at[pl.ds(peoff, CHUNK)], pbuf)
            for b in range(B):
                xoff = pl.multiple_of(b * S * D + row0 * D, CHUNK)
                pltpu.sync_copy(x_hbm.at[pl.ds(xoff, CHUNK)], xbuf)

                @plsc.parallel_loop(0, CHUNK, 16, unroll=8)
                def _(i):
                    o = pl.multiple_of(i, 16)
                    obuf[pl.ds(o, 16)] = xbuf[pl.ds(o, 16)] + pbuf[pl.ds(o, 16)]

                pltpu.sync_copy(obuf, out_hbm.at[pl.ds(xoff, CHUNK)])

    out = sc_add(x1, pe1)
    return out.reshape(B, S, D)


# speedup vs baseline: 1.6769x; 1.2144x over previous
import functools

import jax
import jax.numpy as jnp
from jax import lax
from jax.experimental import pallas as pl
from jax.experimental.pallas import tpu as pltpu
from jax.experimental.pallas import tpu_sc as plsc

NC = 2
NS = 16
NW = NC * NS

ROWS_PER_CHUNK = 16
CH = ROWS_PER_CHUNK * 1024


def kernel(x, pe, position):
    B, S, D = x.shape
    M = pe.shape[0]
    n_elem = B * S * D
    rows_per_w = S // NW
    n_pc = rows_per_w // ROWS_PER_CHUNK
    n_k = n_pc * B
    x1 = x.reshape(n_elem)
    pe1 = pe.reshape(M * D)

    mesh = plsc.VectorSubcoreMesh(core_axis_name="c", subcore_axis_name="s")

    @functools.partial(
        pl.kernel,
        mesh=mesh,
        out_type=jax.ShapeDtypeStruct((n_elem,), jnp.float32),
        scratch_types=[
            pltpu.VMEM((CH,), jnp.float32), pltpu.VMEM((CH,), jnp.float32),
            pltpu.VMEM((CH,), jnp.float32), pltpu.VMEM((CH,), jnp.float32),
            pltpu.VMEM((CH,), jnp.float32), pltpu.VMEM((CH,), jnp.float32),
            pltpu.SemaphoreType.DMA, pltpu.SemaphoreType.DMA,
            pltpu.SemaphoreType.DMA, pltpu.SemaphoreType.DMA,
            pltpu.SemaphoreType.DMA, pltpu.SemaphoreType.DMA,
        ],
    )
    def sc_add(x_hbm, pe_hbm, out_hbm,
               xb0, xb1, ob0, ob1, pb0, pb1,
               sx0, sx1, so0, so1, sp0, sp1):
        wid = lax.axis_index("s") * NC + lax.axis_index("c")
        xb = [xb0, xb1]
        ob = [ob0, ob1]
        pb = [pb0, pb1]
        sx = [sx0, sx1]
        so = [so0, so1]
        sp = [sp0, sp1]
        base = wid * (rows_per_w * D)

        def peoff(pc):
            return pl.multiple_of(base + pc * CH, CH)

        def xoff(k):
            pc, b = divmod(k, B)
            return pl.multiple_of(b * (S * D) + base + pc * CH, CH)

        pltpu.make_async_copy(pe_hbm.at[pl.ds(peoff(0), CH)], pb[0], sp[0]).start()
        pltpu.make_async_copy(x_hbm.at[pl.ds(xoff(0), CH)], xb[0], sx[0]).start()

        for k in range(n_k):
            pc, b = divmod(k, B)
            cur = k % 2
            nxt = (k + 1) % 2
            if k + 1 < n_k:
                pc1, b1 = divmod(k + 1, B)
                if b1 == 0:
                    pltpu.make_async_copy(
                        pe_hbm.at[pl.ds(peoff(pc1), CH)], pb[pc1 % 2], sp[pc1 % 2]
                    ).start()
                pltpu.make_async_copy(
                    x_hbm.at[pl.ds(xoff(k + 1), CH)], xb[nxt], sx[nxt]
                ).start()
            if b == 0:
                pltpu.make_async_copy(
                    pe_hbm.at[pl.ds(peoff(pc), CH)], pb[pc % 2], sp[pc % 2]
                ).wait()
            pltpu.make_async_copy(
                x_hbm.at[pl.ds(xoff(k), CH)], xb[cur], sx[cur]
            ).wait()
            if k >= 2:
                pltpu.make_async_copy(
                    ob[cur], out_hbm.at[pl.ds(xoff(k - 2), CH)], so[cur]
                ).wait()

            xcur = xb[cur]
            ocur = ob[cur]
            pcur = pb[pc % 2]

            @plsc.parallel_loop(0, CH, 16, unroll=8)
            def _(i):
                o = pl.multiple_of(i, 16)
                ocur[pl.ds(o, 16)] = xcur[pl.ds(o, 16)] + pcur[pl.ds(o, 16)]

            pltpu.make_async_copy(
                ocur, out_hbm.at[pl.ds(xoff(k), CH)], so[cur]
            ).start()

        for k in (n_k - 2, n_k - 1):
            pltpu.make_async_copy(
                ob[k % 2], out_hbm.at[pl.ds(xoff(k), CH)], so[k % 2]
            ).wait()

    out = sc_add(x1, pe1)
    return out.reshape(B, S, D)


# speedup vs baseline: 7.0787x; 4.2213x over previous
import jax
import jax.numpy as jnp
from jax.experimental import pallas as pl
from jax.experimental.pallas import tpu as pltpu

S_BLK = 2048


def _add_kernel(pos_ref, x_ref, pe_ref, o_ref):
    o_ref[...] = x_ref[...] + pe_ref[...]


def kernel(x, pe, position):
    B, S, D = x.shape
    n_s = S // S_BLK
    pos32 = position.astype(jnp.int32)

    grid_spec = pltpu.PrefetchScalarGridSpec(
        num_scalar_prefetch=1,
        grid=(n_s, B),
        in_specs=[
            pl.BlockSpec((1, S_BLK, D), lambda i, j, pos: (j, i, 0)),
            pl.BlockSpec((S_BLK, D), lambda i, j, pos: (pos[i * S_BLK] // S_BLK, 0)),
        ],
        out_specs=pl.BlockSpec((1, S_BLK, D), lambda i, j, pos: (j, i, 0)),
    )
    return pl.pallas_call(
        _add_kernel,
        grid_spec=grid_spec,
        out_shape=jax.ShapeDtypeStruct(x.shape, x.dtype),
    )(pos32, x, pe)
